# Initial kernel scaffold; baseline (speedup 1.0000x reference)
import jax
import jax.numpy as jnp
from jax.experimental import pallas as pl


def kernel(atom_in_fea, nbr_fea, nbr_fea_idx, W_nbr, b_nbr, W_in, b_in, bn_gamma, bn_beta):
    raise NotImplementedError("write your pallas kernel here")


# speedup vs baseline: 1.6457x; 1.6457x over previous
import functools

import jax
import jax.numpy as jnp
from jax import lax
from jax.experimental import pallas as pl
from jax.experimental.pallas import tpu as pltpu
from jax.experimental.pallas import tpu_sc as plsc

N = 10000
M = 32
AF = 128
NF = 16

NC = 2
NS = 16
NW = NC * NS

CB = 4
RW = 320
NPAD = NW * RW
NCH = RW // CB

_sc_fn_cache = None


def _sc_gather_sum(idx_pad, table):
    global _sc_fn_cache
    if _sc_fn_cache is None:
        mesh = plsc.VectorSubcoreMesh(
            core_axis_name="c", subcore_axis_name="s",
            num_cores=NC, num_subcores=NS)

        @functools.partial(
            pl.kernel,
            mesh=mesh,
            out_type=jax.ShapeDtypeStruct((NPAD, AF), jnp.float32),
            scratch_types=[
                pltpu.VMEM((CB * M,), jnp.int32),
                pltpu.VMEM((CB * M, AF), jnp.float32),
                pltpu.VMEM((CB, AF), jnp.float32),
                pltpu.SemaphoreType.DMA,
            ],
        )
        def sc_body(idx_hbm, table_hbm, out_hbm, idx_v, rows_v, acc_v, sem):
            wid = lax.axis_index("s") * NC + lax.axis_index("c")

            def chunk_body(ch, _):
                row0 = wid * RW + ch * CB
                pltpu.sync_copy(idx_hbm.at[pl.ds(row0 * M, CB * M)], idx_v)
                pltpu.async_copy(table_hbm.at[idx_v], rows_v, sem).wait()
                for r in range(CB):
                    base = r * M

                    def mbody(m, accs):
                        return tuple(
                            accs[c] + rows_v[base + m, pl.ds(c * 16, 16)]
                            for c in range(8)
                        )

                    accs = tuple(
                        rows_v[base, pl.ds(c * 16, 16)] for c in range(8))
                    accs = lax.fori_loop(1, M, mbody, accs)
                    for c in range(8):
                        acc_v[r, pl.ds(c * 16, 16)] = accs[c]
                pltpu.sync_copy(acc_v, out_hbm.at[pl.ds(row0, CB)])
                return 0

            lax.fori_loop(0, NCH, chunk_body, 0)

        _sc_fn_cache = sc_body
    return _sc_fn_cache(idx_pad, table)


BR = 1000
NB = N // BR


def _tc_body(S_ref, atom_ref, nbr_ref, Wa_ref, Wi_ref, Wb_ref, bias_ref,
             gamma_ref, beta_ref, out_ref, tf_ref, sum_ref, sq_ref):
    p = pl.program_id(0)
    i = pl.program_id(1)

    @pl.when(p == 0)
    def _phase0():
        x = (
            jnp.dot(S_ref[...], Wa_ref[...], preferred_element_type=jnp.float32)
            + jnp.dot(atom_ref[...], Wi_ref[...], preferred_element_type=jnp.float32)
            + jnp.dot(nbr_ref[...], Wb_ref[...], preferred_element_type=jnp.float32)
            + bias_ref[...]
        )
        tf_ref[pl.ds(i * BR, BR), :] = x
        colsum = jnp.sum(x, axis=0, keepdims=True)
        colsq = jnp.sum(x * x, axis=0, keepdims=True)

        @pl.when(i == 0)
        def _():
            sum_ref[...] = colsum
            sq_ref[...] = colsq

        @pl.when(i > 0)
        def _():
            sum_ref[...] = sum_ref[...] + colsum
            sq_ref[...] = sq_ref[...] + colsq

    @pl.when(p == 1)
    def _phase1():
        inv_n = jnp.float32(1.0 / N)
        mean = sum_ref[...] * inv_n
        var = sq_ref[...] * inv_n - mean * mean
        scale = gamma_ref[...] * lax.rsqrt(var + jnp.float32(1e-5))
        shift = beta_ref[...] - mean * scale
        y = tf_ref[pl.ds(i * BR, BR), :] * scale + shift
        out_ref[...] = jnp.maximum(y, 0.0) + jnp.log1p(jnp.exp(-jnp.abs(y)))


def _tc_call(S, atom, nbr2d, Wa, Wi, Wb, bias, gamma, beta):
    full = lambda shape: pl.BlockSpec(shape, lambda p, i: (0, 0))
    return pl.pallas_call(
        _tc_body,
        grid=(2, NB),
        in_specs=[
            pl.BlockSpec((BR, AF), lambda p, i: (jnp.where(p == 0, i, 0), 0)),
            pl.BlockSpec((BR, AF), lambda p, i: (jnp.where(p == 0, i, 0), 0)),
            pl.BlockSpec((BR, M * NF), lambda p, i: (jnp.where(p == 0, i, 0), 0)),
            full((AF, AF)),
            full((AF, AF)),
            full((M * NF, AF)),
            full((1, AF)),
            full((1, AF)),
            full((1, AF)),
        ],
        out_specs=pl.BlockSpec((BR, AF), lambda p, i: (jnp.where(p == 1, i, 0), 0)),
        out_shape=jax.ShapeDtypeStruct((N, AF), jnp.float32),
        scratch_shapes=[
            pltpu.VMEM((N, AF), jnp.float32),
            pltpu.VMEM((1, AF), jnp.float32),
            pltpu.VMEM((1, AF), jnp.float32),
        ],
    )(S, atom, nbr2d, Wa, Wi, Wb, bias, gamma, beta)


def kernel(atom_in_fea, nbr_fea, nbr_fea_idx, W_nbr, b_nbr, W_in, b_in,
           bn_gamma, bn_beta):
    idx = nbr_fea_idx.astype(jnp.int32).reshape(-1)
    idx_pad = jnp.concatenate(
        [idx, jnp.zeros((NPAD * M - N * M,), jnp.int32)])
    S = _sc_gather_sum(idx_pad, atom_in_fea)

    nbr2d = nbr_fea.reshape(N, M * NF)
    Wa = W_nbr[:, :AF].T
    Wi = W_in.T
    Wb = jnp.tile(W_nbr[:, AF:].T, (M, 1))
    bias = (M * b_nbr + b_in).reshape(1, AF)
    return _tc_call(S, atom_in_fea, nbr2d, Wa, Wi, Wb, bias,
                    bn_gamma.reshape(1, AF), bn_beta.reshape(1, AF))


# speedup vs baseline: 1.9201x; 1.1667x over previous
import functools

import jax
import jax.numpy as jnp
from jax import lax
from jax.experimental import pallas as pl
from jax.experimental.pallas import tpu as pltpu
from jax.experimental.pallas import tpu_sc as plsc

N = 10000
M = 32
AF = 128
NF = 16

NC = 2
NS = 16
NW = NC * NS

CB = 4
RW = 320
NPAD = NW * RW
NCH = RW // CB

_sc_fn_cache = None


def _sc_gather_sum(idx_pad, table):
    global _sc_fn_cache
    if _sc_fn_cache is None:
        mesh = plsc.VectorSubcoreMesh(
            core_axis_name="c", subcore_axis_name="s",
            num_cores=NC, num_subcores=NS)

        @functools.partial(
            pl.kernel,
            mesh=mesh,
            out_type=jax.ShapeDtypeStruct((NPAD, AF), jnp.float32),
            scratch_types=[
                pltpu.VMEM((RW * M,), jnp.int32),
                pltpu.VMEM((CB * M, AF), jnp.float32),
                pltpu.VMEM((CB * M, AF), jnp.float32),
                pltpu.VMEM((RW, AF), jnp.float32),
                pltpu.SemaphoreType.DMA,
                pltpu.SemaphoreType.DMA,
            ],
        )
        def sc_body(idx_hbm, table_hbm, out_hbm, idx_all, rows_a, rows_b,
                    out_v, sem_a, sem_b):
            wid = lax.axis_index("s") * NC + lax.axis_index("c")
            base_row = wid * RW
            pltpu.sync_copy(idx_hbm.at[pl.ds(base_row * M, RW * M)], idx_all)

            def start_gather(ch, rows, sem):
                ch = jnp.minimum(ch, NCH - 1)
                idx_slice = idx_all.at[pl.ds(ch * (CB * M), CB * M)]
                pltpu.async_copy(table_hbm.at[idx_slice], rows, sem)

            def wait_rows(rows, sem):
                pltpu.make_async_copy(
                    table_hbm.at[pl.ds(0, CB * M)], rows, sem).wait()

            def reduce_chunk(rows, ch):
                for r in range(CB):
                    row = ch * CB + r
                    for c in range(8):
                        acc = rows[r * M, pl.ds(c * 16, 16)]
                        for m in range(1, M):
                            acc = acc + rows[r * M + m, pl.ds(c * 16, 16)]
                        out_v[row, pl.ds(c * 16, 16)] = acc

            def body2(i, _):
                ch = 2 * i
                start_gather(ch + 1, rows_b, sem_b)
                wait_rows(rows_a, sem_a)
                reduce_chunk(rows_a, ch)
                start_gather(ch + 2, rows_a, sem_a)
                wait_rows(rows_b, sem_b)
                reduce_chunk(rows_b, ch + 1)
                return 0

            start_gather(0, rows_a, sem_a)
            lax.fori_loop(0, NCH // 2, body2, 0)
            wait_rows(rows_a, sem_a)
            pltpu.sync_copy(out_v, out_hbm.at[pl.ds(base_row, RW)])

        _sc_fn_cache = sc_body
    return _sc_fn_cache(idx_pad, table)


BR = 1000
NB = N // BR


def _tc_body(S_ref, atom_ref, nbr_ref, Wa_ref, Wi_ref, Wb_ref, bias_ref,
             gamma_ref, beta_ref, out_ref, tf_ref, sum_ref, sq_ref):
    p = pl.program_id(0)
    i = pl.program_id(1)

    @pl.when(p == 0)
    def _phase0():
        x = (
            jnp.dot(S_ref[...], Wa_ref[...], preferred_element_type=jnp.float32)
            + jnp.dot(atom_ref[...], Wi_ref[...], preferred_element_type=jnp.float32)
            + jnp.dot(nbr_ref[...], Wb_ref[...], preferred_element_type=jnp.float32)
            + bias_ref[...]
        )
        tf_ref[pl.ds(i * BR, BR), :] = x
        colsum = jnp.sum(x, axis=0, keepdims=True)
        colsq = jnp.sum(x * x, axis=0, keepdims=True)

        @pl.when(i == 0)
        def _():
            sum_ref[...] = colsum
            sq_ref[...] = colsq

        @pl.when(i > 0)
        def _():
            sum_ref[...] = sum_ref[...] + colsum
            sq_ref[...] = sq_ref[...] + colsq

    @pl.when(p == 1)
    def _phase1():
        inv_n = jnp.float32(1.0 / N)
        mean = sum_ref[...] * inv_n
        var = sq_ref[...] * inv_n - mean * mean
        scale = gamma_ref[...] * lax.rsqrt(var + jnp.float32(1e-5))
        shift = beta_ref[...] - mean * scale
        y = tf_ref[pl.ds(i * BR, BR), :] * scale + shift
        out_ref[...] = jnp.maximum(y, 0.0) + jnp.log1p(jnp.exp(-jnp.abs(y)))


def _tc_call(S, atom, nbr2d, Wa, Wi, Wb, bias, gamma, beta):
    full = lambda shape: pl.BlockSpec(shape, lambda p, i: (0, 0))
    return pl.pallas_call(
        _tc_body,
        grid=(2, NB),
        in_specs=[
            pl.BlockSpec((BR, AF), lambda p, i: (jnp.where(p == 0, i, 0), 0)),
            pl.BlockSpec((BR, AF), lambda p, i: (jnp.where(p == 0, i, 0), 0)),
            pl.BlockSpec((BR, M * NF), lambda p, i: (jnp.where(p == 0, i, 0), 0)),
            full((AF, AF)),
            full((AF, AF)),
            full((M * NF, AF)),
            full((1, AF)),
            full((1, AF)),
            full((1, AF)),
        ],
        out_specs=pl.BlockSpec((BR, AF), lambda p, i: (jnp.where(p == 1, i, 0), 0)),
        out_shape=jax.ShapeDtypeStruct((N, AF), jnp.float32),
        scratch_shapes=[
            pltpu.VMEM((N, AF), jnp.float32),
            pltpu.VMEM((1, AF), jnp.float32),
            pltpu.VMEM((1, AF), jnp.float32),
        ],
    )(S, atom, nbr2d, Wa, Wi, Wb, bias, gamma, beta)


def kernel(atom_in_fea, nbr_fea, nbr_fea_idx, W_nbr, b_nbr, W_in, b_in,
           bn_gamma, bn_beta):
    idx = nbr_fea_idx.astype(jnp.int32).reshape(-1)
    idx_pad = jnp.concatenate(
        [idx, jnp.zeros((NPAD * M - N * M,), jnp.int32)])
    S = _sc_gather_sum(idx_pad, atom_in_fea)

    nbr2d = nbr_fea.reshape(N, M * NF)
    Wa = W_nbr[:, :AF].T
    Wi = W_in.T
    Wb = jnp.tile(W_nbr[:, AF:].T, (M, 1))
    bias = (M * b_nbr + b_in).reshape(1, AF)
    return _tc_call(S, atom_in_fea, nbr2d, Wa, Wi, Wb, bias,
                    bn_gamma.reshape(1, AF), bn_beta.reshape(1, AF))


# speedup vs baseline: 1.9236x; 1.0018x over previous
import functools

import jax
import jax.numpy as jnp
from jax import lax
from jax.experimental import pallas as pl
from jax.experimental.pallas import tpu as pltpu
from jax.experimental.pallas import tpu_sc as plsc

N = 10000
M = 32
AF = 128
NF = 16

NC = 2
NS = 16
NW = NC * NS

CB = 4
RW = 320
NPAD = NW * RW
NCH = RW // CB
NBUF = 2

_sc_fn_cache = None


def _sc_gather_sum(idx_pad, table):
    global _sc_fn_cache
    if _sc_fn_cache is None:
        mesh = plsc.VectorSubcoreMesh(
            core_axis_name="c", subcore_axis_name="s",
            num_cores=NC, num_subcores=NS)

        @functools.partial(
            pl.kernel,
            mesh=mesh,
            out_type=jax.ShapeDtypeStruct((NPAD, AF), jnp.float32),
            scratch_types=(
                [pltpu.VMEM((RW * M,), jnp.int32)]
                + [pltpu.VMEM((CB * M, AF), jnp.float32) for _ in range(NBUF)]
                + [pltpu.VMEM((RW, AF), jnp.float32)]
                + [pltpu.SemaphoreType.DMA for _ in range(NBUF)]
            ),
        )
        def sc_body(idx_hbm, table_hbm, out_hbm, idx_all, r0, r1,
                    out_v, s0, s1):
            rows = [r0, r1]
            sems = [s0, s1]
            wid = lax.axis_index("s") * NC + lax.axis_index("c")
            base_row = wid * RW
            pltpu.sync_copy(idx_hbm.at[pl.ds(base_row * M, RW * M)], idx_all)

            def start_gather(ch, b):
                ch = jnp.minimum(ch, NCH - 1)
                idx_slice = idx_all.at[pl.ds(ch * (CB * M), CB * M)]
                pltpu.async_copy(table_hbm.at[idx_slice], rows[b], sems[b])

            def wait_rows(b):
                pltpu.make_async_copy(
                    table_hbm.at[pl.ds(0, CB * M)], rows[b], sems[b]).wait()

            def reduce_chunk(b, ch):
                for r in range(CB):
                    row = ch * CB + r
                    for c in range(8):
                        acc = rows[b][r * M, pl.ds(c * 16, 16)]
                        for m in range(1, M):
                            acc = acc + rows[b][r * M + m, pl.ds(c * 16, 16)]
                        out_v[row, pl.ds(c * 16, 16)] = acc

            for b in range(NBUF - 1):
                start_gather(b, b)

            def bodyn(i, _):
                ch = NBUF * i
                for b in range(NBUF):
                    start_gather(ch + b + NBUF - 1, (b + NBUF - 1) % NBUF)
                    wait_rows(b)
                    reduce_chunk(b, ch + b)
                return 0

            lax.fori_loop(0, NCH // NBUF, bodyn, 0)
            for b in range(NBUF - 1):
                wait_rows(b)
            pltpu.sync_copy(out_v, out_hbm.at[pl.ds(base_row, RW)])

        _sc_fn_cache = sc_body
    return _sc_fn_cache(idx_pad, table)


BR = 1000
NB = N // BR


def _tc_body(S_ref, atom_ref, nbr_ref, Wa_ref, Wi_ref, Wb_ref, bias_ref,
             gamma_ref, beta_ref, out_ref, tf_ref, sum_ref, sq_ref):
    p = pl.program_id(0)
    i = pl.program_id(1)

    @pl.when(p == 0)
    def _phase0():
        x = (
            jnp.dot(S_ref[...], Wa_ref[...], preferred_element_type=jnp.float32)
            + jnp.dot(atom_ref[...], Wi_ref[...], preferred_element_type=jnp.float32)
            + jnp.dot(nbr_ref[...], Wb_ref[...], preferred_element_type=jnp.float32)
            + bias_ref[...]
        )
        tf_ref[pl.ds(i * BR, BR), :] = x
        colsum = jnp.sum(x, axis=0, keepdims=True)
        colsq = jnp.sum(x * x, axis=0, keepdims=True)

        @pl.when(i == 0)
        def _():
            sum_ref[...] = colsum
            sq_ref[...] = colsq

        @pl.when(i > 0)
        def _():
            sum_ref[...] = sum_ref[...] + colsum
            sq_ref[...] = sq_ref[...] + colsq

    @pl.when(p == 1)
    def _phase1():
        inv_n = jnp.float32(1.0 / N)
        mean = sum_ref[...] * inv_n
        var = sq_ref[...] * inv_n - mean * mean
        scale = gamma_ref[...] * lax.rsqrt(var + jnp.float32(1e-5))
        shift = beta_ref[...] - mean * scale
        y = tf_ref[pl.ds(i * BR, BR), :] * scale + shift
        out_ref[...] = jnp.maximum(y, 0.0) + jnp.log1p(jnp.exp(-jnp.abs(y)))


def _tc_call(S, atom, nbr2d, Wa, Wi, Wb, bias, gamma, beta):
    full = lambda shape: pl.BlockSpec(shape, lambda p, i: (0, 0))
    return pl.pallas_call(
        _tc_body,
        grid=(2, NB),
        in_specs=[
            pl.BlockSpec((BR, AF), lambda p, i: (jnp.where(p == 0, i, 0), 0)),
            pl.BlockSpec((BR, AF), lambda p, i: (jnp.where(p == 0, i, 0), 0)),
            pl.BlockSpec((BR, M * NF), lambda p, i: (jnp.where(p == 0, i, 0), 0)),
            full((AF, AF)),
            full((AF, AF)),
            full((M * NF, AF)),
            full((1, AF)),
            full((1, AF)),
            full((1, AF)),
        ],
        out_specs=pl.BlockSpec((BR, AF), lambda p, i: (jnp.where(p == 1, i, 0), 0)),
        out_shape=jax.ShapeDtypeStruct((N, AF), jnp.float32),
        scratch_shapes=[
            pltpu.VMEM((N, AF), jnp.float32),
            pltpu.VMEM((1, AF), jnp.float32),
            pltpu.VMEM((1, AF), jnp.float32),
        ],
    )(S, atom, nbr2d, Wa, Wi, Wb, bias, gamma, beta)


def kernel(atom_in_fea, nbr_fea, nbr_fea_idx, W_nbr, b_nbr, W_in, b_in,
           bn_gamma, bn_beta):
    idx = nbr_fea_idx.astype(jnp.int32).reshape(-1)
    idx_pad = jnp.concatenate(
        [idx, jnp.zeros((NPAD * M - N * M,), jnp.int32)])
    S = _sc_gather_sum(idx_pad, atom_in_fea)

    nbr2d = nbr_fea.reshape(N, M * NF)
    Wa = W_nbr[:, :AF].T
    Wi = W_in.T
    Wb = jnp.tile(W_nbr[:, AF:].T, (M, 1))
    bias = (M * b_nbr + b_in).reshape(1, AF)
    return _tc_call(S, atom_in_fea, nbr2d, Wa, Wi, Wb, bias,
                    bn_gamma.reshape(1, AF), bn_beta.reshape(1, AF))


# speedup vs baseline: 2.0604x; 1.0711x over previous
import functools

import jax
import jax.numpy as jnp
from jax import lax
from jax.experimental import pallas as pl
from jax.experimental.pallas import tpu as pltpu
from jax.experimental.pallas import tpu_sc as plsc

N = 10000
M = 32
AF = 128
NF = 16

NC = 2
NS = 16
NW = NC * NS

CB = 4
RW0 = 400
RW1 = 240
RWMAX = max(RW0, RW1)
NPAD = NS * (RW0 + RW1)
NCH0 = RW0 // CB
NCH1 = RW1 // CB
NBUF = 2

_sc_fn_cache = None


def _sc_gather_sum(idx_pad, table):
    global _sc_fn_cache
    if _sc_fn_cache is None:
        mesh = plsc.VectorSubcoreMesh(
            core_axis_name="c", subcore_axis_name="s",
            num_cores=NC, num_subcores=NS)

        @functools.partial(
            pl.kernel,
            mesh=mesh,
            out_type=jax.ShapeDtypeStruct((NPAD, AF), jnp.float32),
            scratch_types=(
                [pltpu.VMEM((RWMAX * M,), jnp.int32)]
                + [pltpu.VMEM((CB * M, AF), jnp.float32) for _ in range(NBUF)]
                + [pltpu.VMEM((RWMAX, AF), jnp.float32)]
                + [pltpu.SemaphoreType.DMA for _ in range(NBUF)]
            ),
        )
        def sc_body(idx_hbm, table_hbm, out_hbm, idx_all, r0, r1,
                    out_v, s0, s1):
            rows = [r0, r1]
            sems = [s0, s1]
            cid = lax.axis_index("c")
            sid = lax.axis_index("s")
            base_row = sid * (RW0 + RW1) + cid * RW0
            rw = jnp.where(cid == 0, RW0, RW1)
            nch = jnp.where(cid == 0, NCH0, NCH1)
            pltpu.sync_copy(idx_hbm.at[pl.ds(base_row * M, RW1 * M)],
                            idx_all.at[pl.ds(0, RW1 * M)])

            @pl.when(cid == 0)
            def _extra_idx():
                pltpu.sync_copy(
                    idx_hbm.at[pl.ds(base_row * M + RW1 * M,
                                     (RW0 - RW1) * M)],
                    idx_all.at[pl.ds(RW1 * M, (RW0 - RW1) * M)])

            def start_gather(ch, b):
                ch = jnp.minimum(ch, nch - 1)
                idx_slice = idx_all.at[pl.ds(ch * (CB * M), CB * M)]
                pltpu.async_copy(table_hbm.at[idx_slice], rows[b], sems[b])

            def wait_rows(b):
                pltpu.make_async_copy(
                    table_hbm.at[pl.ds(0, CB * M)], rows[b], sems[b]).wait()

            def reduce_chunk(b, ch):
                for r in range(CB):
                    row = ch * CB + r
                    for c in range(8):
                        acc = rows[b][r * M, pl.ds(c * 16, 16)]
                        for m in range(1, M):
                            acc = acc + rows[b][r * M + m, pl.ds(c * 16, 16)]
                        out_v[row, pl.ds(c * 16, 16)] = acc

            for b in range(NBUF - 1):
                start_gather(b, b)

            def bodyn(i, _):
                ch = NBUF * i
                for b in range(NBUF):
                    start_gather(ch + b + NBUF - 1, (b + NBUF - 1) % NBUF)
                    wait_rows(b)
                    reduce_chunk(b, ch + b)
                return 0

            lax.fori_loop(0, nch // NBUF, bodyn, 0)
            for b in range(NBUF - 1):
                wait_rows(b)
            pltpu.sync_copy(out_v.at[pl.ds(0, RW1)],
                            out_hbm.at[pl.ds(base_row, RW1)])

            @pl.when(cid == 0)
            def _extra_out():
                pltpu.sync_copy(
                    out_v.at[pl.ds(RW1, RW0 - RW1)],
                    out_hbm.at[pl.ds(base_row + RW1, RW0 - RW1)])

        _sc_fn_cache = sc_body
    return _sc_fn_cache(idx_pad, table)


BR = 1000
NB = N // BR


def _tc_body(S_ref, atom_ref, nbr_ref, Wa_ref, Wi_ref, Wb_ref, bias_ref,
             gamma_ref, beta_ref, out_ref, tf_ref, sum_ref, sq_ref):
    p = pl.program_id(0)
    i = pl.program_id(1)

    @pl.when(p == 0)
    def _phase0():
        x = (
            jnp.dot(S_ref[...], Wa_ref[...], preferred_element_type=jnp.float32)
            + jnp.dot(atom_ref[...], Wi_ref[...], preferred_element_type=jnp.float32)
            + jnp.dot(nbr_ref[...], Wb_ref[...], preferred_element_type=jnp.float32)
            + bias_ref[...]
        )
        tf_ref[pl.ds(i * BR, BR), :] = x
        colsum = jnp.sum(x, axis=0, keepdims=True)
        colsq = jnp.sum(x * x, axis=0, keepdims=True)

        @pl.when(i == 0)
        def _():
            sum_ref[...] = colsum
            sq_ref[...] = colsq

        @pl.when(i > 0)
        def _():
            sum_ref[...] = sum_ref[...] + colsum
            sq_ref[...] = sq_ref[...] + colsq

    @pl.when(p == 1)
    def _phase1():
        inv_n = jnp.float32(1.0 / N)
        mean = sum_ref[...] * inv_n
        var = sq_ref[...] * inv_n - mean * mean
        scale = gamma_ref[...] * lax.rsqrt(var + jnp.float32(1e-5))
        shift = beta_ref[...] - mean * scale
        y = tf_ref[pl.ds(i * BR, BR), :] * scale + shift
        out_ref[...] = jnp.maximum(y, 0.0) + jnp.log1p(jnp.exp(-jnp.abs(y)))


def _tc_call(S, atom, nbr2d, Wa, Wi, Wb, bias, gamma, beta):
    full = lambda shape: pl.BlockSpec(shape, lambda p, i: (0, 0))
    return pl.pallas_call(
        _tc_body,
        grid=(2, NB),
        in_specs=[
            pl.BlockSpec((BR, AF), lambda p, i: (jnp.where(p == 0, i, 0), 0)),
            pl.BlockSpec((BR, AF), lambda p, i: (jnp.where(p == 0, i, 0), 0)),
            pl.BlockSpec((BR, M * NF), lambda p, i: (jnp.where(p == 0, i, 0), 0)),
            full((AF, AF)),
            full((AF, AF)),
            full((M * NF, AF)),
            full((1, AF)),
            full((1, AF)),
            full((1, AF)),
        ],
        out_specs=pl.BlockSpec((BR, AF), lambda p, i: (jnp.where(p == 1, i, 0), 0)),
        out_shape=jax.ShapeDtypeStruct((N, AF), jnp.float32),
        scratch_shapes=[
            pltpu.VMEM((N, AF), jnp.float32),
            pltpu.VMEM((1, AF), jnp.float32),
            pltpu.VMEM((1, AF), jnp.float32),
        ],
    )(S, atom, nbr2d, Wa, Wi, Wb, bias, gamma, beta)


def kernel(atom_in_fea, nbr_fea, nbr_fea_idx, W_nbr, b_nbr, W_in, b_in,
           bn_gamma, bn_beta):
    idx = nbr_fea_idx.astype(jnp.int32).reshape(-1)
    idx_pad = jnp.concatenate(
        [idx, jnp.zeros((NPAD * M - N * M,), jnp.int32)])
    S = _sc_gather_sum(idx_pad, atom_in_fea)

    nbr2d = nbr_fea.reshape(N, M * NF)
    Wa = W_nbr[:, :AF].T
    Wi = W_in.T
    Wb = jnp.tile(W_nbr[:, AF:].T, (M, 1))
    bias = (M * b_nbr + b_in).reshape(1, AF)
    return _tc_call(S, atom_in_fea, nbr2d, Wa, Wi, Wb, bias,
                    bn_gamma.reshape(1, AF), bn_beta.reshape(1, AF))


# speedup vs baseline: 2.1285x; 1.0330x over previous
import functools

import jax
import jax.numpy as jnp
from jax import lax
from jax.experimental import pallas as pl
from jax.experimental.pallas import tpu as pltpu
from jax.experimental.pallas import tpu_sc as plsc

N = 10000
M = 32
AF = 128
NF = 16

NC = 2
NS = 16
NW = NC * NS

CB = 4
RW0 = 440
RW1 = 200
RWMAX = max(RW0, RW1)
NPAD = NS * (RW0 + RW1)
NCH0 = RW0 // CB
NCH1 = RW1 // CB
NBUF = 2

_sc_fn_cache = None


def _sc_gather_sum(idx_pad, table):
    global _sc_fn_cache
    if _sc_fn_cache is None:
        mesh = plsc.VectorSubcoreMesh(
            core_axis_name="c", subcore_axis_name="s",
            num_cores=NC, num_subcores=NS)

        @functools.partial(
            pl.kernel,
            mesh=mesh,
            out_type=jax.ShapeDtypeStruct((NPAD, AF), jnp.float32),
            scratch_types=(
                [pltpu.VMEM((RWMAX * M,), jnp.int32)]
                + [pltpu.VMEM((CB * M, AF), jnp.float32) for _ in range(NBUF)]
                + [pltpu.VMEM((RWMAX, AF), jnp.float32)]
                + [pltpu.SemaphoreType.DMA for _ in range(NBUF)]
            ),
        )
        def sc_body(idx_hbm, table_hbm, out_hbm, idx_all, r0, r1,
                    out_v, s0, s1):
            rows = [r0, r1]
            sems = [s0, s1]
            cid = lax.axis_index("c")
            sid = lax.axis_index("s")
            base_row = sid * (RW0 + RW1) + cid * RW0
            rw = jnp.where(cid == 0, RW0, RW1)
            nch = jnp.where(cid == 0, NCH0, NCH1)
            pltpu.sync_copy(idx_hbm.at[pl.ds(base_row * M, RW1 * M)],
                            idx_all.at[pl.ds(0, RW1 * M)])

            @pl.when(cid == 0)
            def _extra_idx():
                pltpu.sync_copy(
                    idx_hbm.at[pl.ds(base_row * M + RW1 * M,
                                     (RW0 - RW1) * M)],
                    idx_all.at[pl.ds(RW1 * M, (RW0 - RW1) * M)])

            def start_gather(ch, b):
                ch = jnp.minimum(ch, nch - 1)
                idx_slice = idx_all.at[pl.ds(ch * (CB * M), CB * M)]
                pltpu.async_copy(table_hbm.at[idx_slice], rows[b], sems[b])

            def wait_rows(b):
                pltpu.make_async_copy(
                    table_hbm.at[pl.ds(0, CB * M)], rows[b], sems[b]).wait()

            def reduce_chunk(b, ch):
                for r in range(CB):
                    row = ch * CB + r
                    for c in range(8):
                        acc = rows[b][r * M, pl.ds(c * 16, 16)]
                        for m in range(1, M):
                            acc = acc + rows[b][r * M + m, pl.ds(c * 16, 16)]
                        out_v[row, pl.ds(c * 16, 16)] = acc

            for b in range(NBUF - 1):
                start_gather(b, b)

            def bodyn(i, _):
                ch = NBUF * i
                for b in range(NBUF):
                    start_gather(ch + b + NBUF - 1, (b + NBUF - 1) % NBUF)
                    wait_rows(b)
                    reduce_chunk(b, ch + b)
                return 0

            lax.fori_loop(0, nch // NBUF, bodyn, 0)
            for b in range(NBUF - 1):
                wait_rows(b)
            pltpu.sync_copy(out_v.at[pl.ds(0, RW1)],
                            out_hbm.at[pl.ds(base_row, RW1)])

            @pl.when(cid == 0)
            def _extra_out():
                pltpu.sync_copy(
                    out_v.at[pl.ds(RW1, RW0 - RW1)],
                    out_hbm.at[pl.ds(base_row + RW1, RW0 - RW1)])

        _sc_fn_cache = sc_body
    return _sc_fn_cache(idx_pad, table)


BR = 1000
NB = N // BR


def _tc_body(S_ref, atom_ref, nbr_ref, Wa_ref, Wi_ref, Wb_ref, bias_ref,
             gamma_ref, beta_ref, out_ref, tf_ref, sum_ref, sq_ref):
    p = pl.program_id(0)
    i = pl.program_id(1)

    @pl.when(p == 0)
    def _phase0():
        x = (
            jnp.dot(S_ref[...], Wa_ref[...], preferred_element_type=jnp.float32)
            + jnp.dot(atom_ref[...], Wi_ref[...], preferred_element_type=jnp.float32)
            + jnp.dot(nbr_ref[...], Wb_ref[...], preferred_element_type=jnp.float32)
            + bias_ref[...]
        )
        tf_ref[pl.ds(i * BR, BR), :] = x
        colsum = jnp.sum(x, axis=0, keepdims=True)
        colsq = jnp.sum(x * x, axis=0, keepdims=True)

        @pl.when(i == 0)
        def _():
            sum_ref[...] = colsum
            sq_ref[...] = colsq

        @pl.when(i > 0)
        def _():
            sum_ref[...] = sum_ref[...] + colsum
            sq_ref[...] = sq_ref[...] + colsq

    @pl.when(p == 1)
    def _phase1():
        inv_n = jnp.float32(1.0 / N)
        mean = sum_ref[...] * inv_n
        var = sq_ref[...] * inv_n - mean * mean
        scale = gamma_ref[...] * lax.rsqrt(var + jnp.float32(1e-5))
        shift = beta_ref[...] - mean * scale
        y = tf_ref[pl.ds(i * BR, BR), :] * scale + shift
        out_ref[...] = jnp.maximum(y, 0.0) + jnp.log1p(jnp.exp(-jnp.abs(y)))


def _tc_call(S, atom, nbr2d, Wa, Wi, Wb, bias, gamma, beta):
    full = lambda shape: pl.BlockSpec(shape, lambda p, i: (0, 0))
    return pl.pallas_call(
        _tc_body,
        grid=(2, NB),
        in_specs=[
            pl.BlockSpec((BR, AF), lambda p, i: (jnp.where(p == 0, i, 0), 0)),
            pl.BlockSpec((BR, AF), lambda p, i: (jnp.where(p == 0, i, 0), 0)),
            pl.BlockSpec((BR, M * NF), lambda p, i: (jnp.where(p == 0, i, 0), 0)),
            full((AF, AF)),
            full((AF, AF)),
            full((M * NF, AF)),
            full((1, AF)),
            full((1, AF)),
            full((1, AF)),
        ],
        out_specs=pl.BlockSpec((BR, AF), lambda p, i: (jnp.where(p == 1, i, 0), 0)),
        out_shape=jax.ShapeDtypeStruct((N, AF), jnp.float32),
        scratch_shapes=[
            pltpu.VMEM((N, AF), jnp.float32),
            pltpu.VMEM((1, AF), jnp.float32),
            pltpu.VMEM((1, AF), jnp.float32),
        ],
    )(S, atom, nbr2d, Wa, Wi, Wb, bias, gamma, beta)


def kernel(atom_in_fea, nbr_fea, nbr_fea_idx, W_nbr, b_nbr, W_in, b_in,
           bn_gamma, bn_beta):
    idx = nbr_fea_idx.astype(jnp.int32).reshape(-1)
    idx_pad = jnp.concatenate(
        [idx, jnp.zeros((NPAD * M - N * M,), jnp.int32)])
    S = _sc_gather_sum(idx_pad, atom_in_fea)

    nbr2d = nbr_fea.reshape(N, M * NF)
    Wa = W_nbr[:, :AF].T
    Wi = W_in.T
    Wb = jnp.tile(W_nbr[:, AF:].T, (M, 1))
    bias = (M * b_nbr + b_in).reshape(1, AF)
    return _tc_call(S, atom_in_fea, nbr2d, Wa, Wi, Wb, bias,
                    bn_gamma.reshape(1, AF), bn_beta.reshape(1, AF))
